# baseline (speedup 1.0000x reference)
import functools

import jax
import jax.numpy as jnp
from jax import lax
from jax.experimental import pallas as pl
from jax.experimental.pallas import tpu as pltpu
from jax.experimental.pallas import tpu_sc as plsc

B = 1; T = 2048; D = 1024; H = 16; KV = 8; HD = 64; L = 2; E = 8; TOPK = 2
FF = 2048; V = 32000; EPS = 1e-5; THETA = 1e6

TB = 256
NTB = T // TB
QH = 4
VB = 1280
NVB = V // VB
A = T * TOPK
EB = 256
NEB = A // EB + E
AP = NEB * EB


def _sc_gather(table, idx):
    info = plsc.get_sparse_core_info()
    nc, ns = info.num_cores, info.num_subcores
    nw = nc * ns
    b, d = idx.shape[0], table.shape[1]
    b_per_w = b // nw
    row_bytes = d * table.dtype.itemsize
    nbuf = 2
    max_ch = (460 * 1024) // (nbuf * row_bytes)
    ch = b_per_w
    for cand in (96, 64, 48, 32, 24, 16, 8):
        if cand <= max_ch and b_per_w % cand == 0:
            ch = cand
            break
    n_ch = b_per_w // ch
    nbuf = min(nbuf, n_ch)
    mesh = plsc.VectorSubcoreMesh(core_axis_name="c", subcore_axis_name="s")

    @functools.partial(
        pl.kernel, mesh=mesh,
        out_type=jax.ShapeDtypeStruct((b, d), table.dtype),
        scratch_types=[
            pltpu.VMEM((n_ch, ch), jnp.int32),
            pltpu.VMEM((nbuf, ch, d), table.dtype),
            pltpu.SemaphoreType.DMA,
        ],
    )
    def k(table_hbm, idx_hbm, out_hbm, idx_v, rows_v, sem):
        wid = lax.axis_index("s") * nc + lax.axis_index("c")
        base = wid * b_per_w
        for c in range(n_ch):
            pltpu.sync_copy(idx_hbm.at[pl.ds(base + c * ch, ch)],
                            idx_v.at[c])
        cps = [None] * n_ch
        for c in range(nbuf):
            cps[c] = pltpu.async_copy(table_hbm.at[idx_v.at[c]],
                                      rows_v.at[c % nbuf], sem)
        for c in range(n_ch):
            cps[c].wait()
            pltpu.sync_copy(rows_v.at[c % nbuf],
                            out_hbm.at[pl.ds(base + c * ch, ch)])
            if c + nbuf < n_ch:
                cps[c + nbuf] = pltpu.async_copy(
                    table_hbm.at[idx_v.at[c + nbuf]],
                    rows_v.at[(c + nbuf) % nbuf], sem)

    return k(table, idx)


def _rms_rows(x, w):
    v = jnp.mean(x * x, axis=-1, keepdims=True)
    return x * lax.rsqrt(v + EPS) * w


def _qkv_body(x_ref, ln_ref, wq_ref, wk_ref, wv_ref, q_ref, k_ref, v_ref):
    h = _rms_rows(x_ref[...], ln_ref[0])
    q_ref[...] = jnp.dot(h, wq_ref[0], preferred_element_type=jnp.float32)
    k_ref[...] = jnp.dot(h, wk_ref[0], preferred_element_type=jnp.float32)
    v_ref[...] = jnp.dot(h, wv_ref[0], preferred_element_type=jnp.float32)


def _rope(x, cos, sin):
    h = HD // 2
    rot = jnp.concatenate([-x[:, h:], x[:, :h]], axis=1)
    return x * cos + rot * sin


KB = 512
NKB = T // KB


def _attn_body(q_ref, k_ref, v_ref, cq_ref, sq_ref, ck_ref, sk_ref, o_ref,
               m_s, l_s, acc_s):
    qi = pl.program_id(1)
    ki = pl.program_id(2)

    @pl.when(ki == 0)
    def _():
        m_s[...] = jnp.full((TB, QH), -1e30, jnp.float32)
        l_s[...] = jnp.zeros((TB, QH), jnp.float32)
        acc_s[...] = jnp.zeros((TB, QH * HD), jnp.float32)

    @pl.when(ki * KB <= qi * TB + TB - 1)
    def _():
        row = qi * TB + lax.broadcasted_iota(jnp.int32, (TB, KB), 0)
        col = ki * KB + lax.broadcasted_iota(jnp.int32, (TB, KB), 1)
        mask = jnp.where(col <= row, 0.0, -1e9)
        kr = [_rope(k_ref[:, j * HD:(j + 1) * HD], ck_ref[...], sk_ref[...])
              for j in range(QH // 2)]
        for j in range(QH):
            q = _rope(q_ref[:, j * HD:(j + 1) * HD], cq_ref[...], sq_ref[...])
            s = lax.dot_general(q, kr[j // 2], (((1,), (1,)), ((), ())),
                                preferred_element_type=jnp.float32)
            s = s * (1.0 / 8.0) + mask
            m_old = m_s[:, j:j + 1]
            m_new = jnp.maximum(m_old, jnp.max(s, axis=1, keepdims=True))
            corr = jnp.exp(m_old - m_new)
            p = jnp.exp(s - m_new)
            l_s[:, j:j + 1] = l_s[:, j:j + 1] * corr + jnp.sum(
                p, axis=1, keepdims=True)
            acc_s[:, j * HD:(j + 1) * HD] = (
                acc_s[:, j * HD:(j + 1) * HD] * corr
                + jnp.dot(p, v_ref[:, (j // 2) * HD:(j // 2 + 1) * HD],
                          preferred_element_type=jnp.float32))
            m_s[:, j:j + 1] = m_new

    @pl.when(ki == NKB - 1)
    def _():
        o_ref[...] = jnp.concatenate(
            [acc_s[:, j * HD:(j + 1) * HD] / l_s[:, j:j + 1]
             for j in range(QH)], axis=1)


def _outproj_body(x_ref, o_ref, wo_ref, y_ref):
    y_ref[...] = x_ref[...] + jnp.dot(o_ref[...], wo_ref[0],
                                      preferred_element_type=jnp.float32)


def _router_body(x_ref, ln_ref, g_ref, h_ref, ti_ref, tv_ref):
    h = _rms_rows(x_ref[...], ln_ref[0])
    h_ref[...] = h
    gl = jnp.dot(h, g_ref[0], preferred_element_type=jnp.float32)
    lane = lax.broadcasted_iota(jnp.int32, (TB, 128), 1)
    gl = jnp.where(lane < E, gl, -1e30)
    m = jnp.max(gl, axis=1, keepdims=True)
    e = jnp.exp(gl - m)
    p = e / jnp.sum(e, axis=1, keepdims=True)
    m1 = jnp.max(p, axis=1, keepdims=True)
    i1 = jnp.min(jnp.where(p == m1, lane, 9999), axis=1, keepdims=True)
    p2 = jnp.where(lane == i1, -1.0, p)
    m2 = jnp.max(p2, axis=1, keepdims=True)
    i2 = jnp.min(jnp.where(p2 == m2, lane, 9999), axis=1, keepdims=True)
    tot = m1 + m2
    ti_ref[...] = jnp.where(lane == 0, i1, jnp.where(lane == 1, i2, 0))
    tv_ref[...] = jnp.where(lane == 0, m1 / tot,
                            jnp.where(lane == 1, m2 / tot, 0.0))


def _expert_body(be_ref, hs_ref, w1_ref, w2_ref, w3_ref, wv_ref, os_ref):
    h = hs_ref[...]
    a = jax.nn.silu(jnp.dot(h, w1_ref[0, 0],
                            preferred_element_type=jnp.float32))
    a = a * jnp.dot(h, w3_ref[0, 0], preferred_element_type=jnp.float32)
    o = jnp.dot(a, w2_ref[0, 0], preferred_element_type=jnp.float32)
    os_ref[...] = o * wv_ref[...]


def _add3_body(x_ref, a_ref, b_ref, y_ref):
    y_ref[...] = x_ref[...] + a_ref[...] + b_ref[...]


def _ce_body(xf_ref, fn_ref, w_ref, lab_ref, out_ref, m_s, s_s, l_s):
    j = pl.program_id(0)

    @pl.when(j == 0)
    def _():
        m_s[...] = jnp.full((T, 1), -1e30, jnp.float32)
        s_s[...] = jnp.zeros((T, 1), jnp.float32)
        l_s[...] = jnp.zeros((T, 1), jnp.float32)

    xn = _rms_rows(xf_ref[...], fn_ref[...])
    lg = jnp.dot(xn, w_ref[...], preferred_element_type=jnp.float32)
    bm = jnp.max(lg, axis=1, keepdims=True)
    m_old = m_s[...]
    m_new = jnp.maximum(m_old, bm)
    s_s[...] = s_s[...] * jnp.exp(m_old - m_new) + jnp.sum(
        jnp.exp(lg - m_new), axis=1, keepdims=True)
    m_s[...] = m_new
    cols = j * VB + lax.broadcasted_iota(jnp.int32, (T, VB), 1)
    match = cols == lab_ref[...]
    l_s[...] = l_s[...] + jnp.sum(jnp.where(match, lg, 0.0), axis=1,
                                  keepdims=True)

    @pl.when(j == NVB - 1)
    def _():
        lse = m_s[...] + jnp.log(s_s[...])
        out_ref[0, 0] = jnp.mean(lse - l_s[...])


_f32 = jnp.float32


def _qkv_call(l, x, ln, wq, wk, wv):
    return pl.pallas_call(
        _qkv_body,
        grid=(NTB,),
        in_specs=[
            pl.BlockSpec((TB, D), lambda i: (i, 0)),
            pl.BlockSpec((1, 1, D), lambda i: (l, 0, 0)),
            pl.BlockSpec((1, D, H * HD), lambda i: (l, 0, 0)),
            pl.BlockSpec((1, D, KV * HD), lambda i: (l, 0, 0)),
            pl.BlockSpec((1, D, KV * HD), lambda i: (l, 0, 0)),
        ],
        out_specs=[
            pl.BlockSpec((TB, H * HD), lambda i: (i, 0)),
            pl.BlockSpec((TB, KV * HD), lambda i: (i, 0)),
            pl.BlockSpec((TB, KV * HD), lambda i: (i, 0)),
        ],
        out_shape=[
            jax.ShapeDtypeStruct((T, H * HD), _f32),
            jax.ShapeDtypeStruct((T, KV * HD), _f32),
            jax.ShapeDtypeStruct((T, KV * HD), _f32),
        ],
    )(x, ln, wq, wk, wv)


def _attn_call(q, k, v, cos, sin):
    return pl.pallas_call(
        _attn_body,
        grid=(H // QH, NTB, NKB),
        in_specs=[
            pl.BlockSpec((TB, QH * HD), lambda h4, qi, ki: (qi, h4)),
            pl.BlockSpec((KB, QH * HD // 2), lambda h4, qi, ki: (ki, h4)),
            pl.BlockSpec((KB, QH * HD // 2), lambda h4, qi, ki: (ki, h4)),
            pl.BlockSpec((TB, HD), lambda h4, qi, ki: (qi, 0)),
            pl.BlockSpec((TB, HD), lambda h4, qi, ki: (qi, 0)),
            pl.BlockSpec((KB, HD), lambda h4, qi, ki: (ki, 0)),
            pl.BlockSpec((KB, HD), lambda h4, qi, ki: (ki, 0)),
        ],
        out_specs=pl.BlockSpec((TB, QH * HD), lambda h4, qi, ki: (qi, h4)),
        out_shape=jax.ShapeDtypeStruct((T, H * HD), _f32),
        scratch_shapes=[
            pltpu.VMEM((TB, QH), jnp.float32),
            pltpu.VMEM((TB, QH), jnp.float32),
            pltpu.VMEM((TB, QH * HD), jnp.float32),
        ],
        compiler_params=pltpu.CompilerParams(
            dimension_semantics=("parallel", "parallel", "arbitrary")),
    )(q, k, v, cos, sin, cos, sin)


def _outproj_call(l, x, o, wo):
    return pl.pallas_call(
        _outproj_body,
        grid=(NTB,),
        in_specs=[
            pl.BlockSpec((TB, D), lambda i: (i, 0)),
            pl.BlockSpec((TB, H * HD), lambda i: (i, 0)),
            pl.BlockSpec((1, H * HD, D), lambda i: (l, 0, 0)),
        ],
        out_specs=pl.BlockSpec((TB, D), lambda i: (i, 0)),
        out_shape=jax.ShapeDtypeStruct((T, D), _f32),
    )(x, o, wo)


def _router_call(l, x, ln, gate_pad):
    return pl.pallas_call(
        _router_body,
        grid=(NTB,),
        in_specs=[
            pl.BlockSpec((TB, D), lambda i: (i, 0)),
            pl.BlockSpec((1, 1, D), lambda i: (l, 0, 0)),
            pl.BlockSpec((1, D, 128), lambda i: (l, 0, 0)),
        ],
        out_specs=[
            pl.BlockSpec((TB, D), lambda i: (i, 0)),
            pl.BlockSpec((TB, 128), lambda i: (i, 0)),
            pl.BlockSpec((TB, 128), lambda i: (i, 0)),
        ],
        out_shape=[
            jax.ShapeDtypeStruct((T, D), _f32),
            jax.ShapeDtypeStruct((T, 128), jnp.int32),
            jax.ShapeDtypeStruct((T, 128), _f32),
        ],
    )(x, ln, gate_pad)


def _expert_call(l, be, hs, w1, w2, w3, wvec):
    spec = pltpu.PrefetchScalarGridSpec(
        num_scalar_prefetch=1,
        grid=(NEB,),
        in_specs=[
            pl.BlockSpec((EB, D), lambda b, be_ref: (b, 0)),
            pl.BlockSpec((1, 1, D, FF),
                         lambda b, be_ref: (l, be_ref[b], 0, 0)),
            pl.BlockSpec((1, 1, FF, D),
                         lambda b, be_ref: (l, be_ref[b], 0, 0)),
            pl.BlockSpec((1, 1, D, FF),
                         lambda b, be_ref: (l, be_ref[b], 0, 0)),
            pl.BlockSpec((EB, 1), lambda b, be_ref: (b, 0)),
        ],
        out_specs=pl.BlockSpec((EB, D), lambda b, be_ref: (b, 0)),
    )
    return pl.pallas_call(
        _expert_body,
        grid_spec=spec,
        out_shape=jax.ShapeDtypeStruct((AP, D), _f32),
    )(be, hs, w1, w2, w3, wvec)


def _add3_call(x, a, b):
    return pl.pallas_call(
        _add3_body,
        grid=(NTB,),
        in_specs=[pl.BlockSpec((TB, D), lambda i: (i, 0))] * 3,
        out_specs=pl.BlockSpec((TB, D), lambda i: (i, 0)),
        out_shape=jax.ShapeDtypeStruct((T, D), _f32),
    )(x, a, b)


def _ce_call(xf, fnorm, lm_head, labels):
    out = pl.pallas_call(
        _ce_body,
        grid=(NVB,),
        in_specs=[
            pl.BlockSpec((T, D), lambda j: (0, 0)),
            pl.BlockSpec((1, D), lambda j: (0, 0)),
            pl.BlockSpec((D, VB), lambda j: (0, j)),
            pl.BlockSpec((T, 1), lambda j: (0, 0)),
        ],
        out_specs=pl.BlockSpec(memory_space=pltpu.SMEM),
        out_shape=jax.ShapeDtypeStruct((1, 1), _f32),
        scratch_shapes=[
            pltpu.VMEM((T, 1), _f32),
            pltpu.VMEM((T, 1), _f32),
            pltpu.VMEM((T, 1), _f32),
        ],
        compiler_params=pltpu.CompilerParams(
            dimension_semantics=("arbitrary",)),
    )(xf, fnorm, lm_head, labels)
    return out[0, 0]


def _route_plan(ti, tv):
    er = jnp.arange(E)
    oh1 = (ti[:, 0:1] == er[None, :]).astype(jnp.int32)
    oh2 = (ti[:, 1:2] == er[None, :]).astype(jnp.int32)
    c1 = jnp.cumsum(oh1, axis=0)
    c2 = jnp.cumsum(oh2, axis=0)
    cnt1 = c1[-1]
    cnt = cnt1 + c2[-1]
    pc = ((cnt + EB - 1) // EB) * EB
    po = jnp.cumsum(pc) - pc
    r1 = jnp.sum((c1 - oh1 + po[None, :]) * oh1, axis=1)
    r2 = jnp.sum((c2 - oh2 + cnt1[None, :] + po[None, :]) * oh2, axis=1)
    tok = jnp.arange(T, dtype=jnp.int32)
    gidx = jnp.zeros((AP,), jnp.int32).at[r1].set(tok).at[r2].set(tok)
    wvec = jnp.zeros((AP,), jnp.float32).at[r1].set(tv[:, 0]).at[r2].set(
        tv[:, 1])
    cidx = jnp.concatenate([r1, r2]).astype(jnp.int32)

    cum_pc = jnp.cumsum(pc)
    s = jnp.arange(NEB) * EB
    be = jnp.minimum(jnp.sum(
        (s[:, None] >= cum_pc[None, :]).astype(jnp.int32), axis=1),
        E - 1).astype(jnp.int32)
    return gidx, wvec.reshape(AP, 1), cidx, be


def kernel(input_ids, labels, embed, ln1, ln2, final_norm, wq, wk, wv, wo,
           gate, w1, w2, w3, lm_head):
    ids = input_ids.reshape(T).astype(jnp.int32)
    x = _sc_gather(embed, ids)

    pos = jnp.arange(T, dtype=jnp.float32)
    inv_f = 1.0 / (THETA ** (jnp.arange(0, HD, 2, dtype=jnp.float32) / HD))
    fr = pos[:, None] * inv_f[None, :]
    emb = jnp.concatenate([fr, fr], axis=-1)
    cos = jnp.cos(emb)
    sin = jnp.sin(emb)

    gate_pad = jnp.pad(gate, ((0, 0), (0, 0), (0, 128 - E)))
    ln1 = ln1.reshape(L, 1, D)
    ln2 = ln2.reshape(L, 1, D)
    for l in range(L):
        q, k, v = _qkv_call(l, x, ln1, wq, wk, wv)
        o = _attn_call(q, k, v, cos, sin)
        x = _outproj_call(l, x, o, wo)

        h, ti_p, tv_p = _router_call(l, x, ln2, gate_pad)
        ti = ti_p[:, :TOPK]
        tv = tv_p[:, :TOPK]
        gidx, wvec, cidx, be = _route_plan(ti, tv)

        hs = _sc_gather(h, gidx)
        os_ = _expert_call(l, be, hs, w1, w2, w3, wvec)
        mo = _sc_gather(os_, cidx)
        x = _add3_call(x, mo[:T], mo[T:])

    lab = labels.reshape(T, 1).astype(jnp.int32)
    return _ce_call(x, final_norm.reshape(1, D), lm_head, lab)


# speedup vs baseline: 1.2817x; 1.2817x over previous
import functools

import jax
import jax.numpy as jnp
from jax import lax
from jax.experimental import pallas as pl
from jax.experimental.pallas import tpu as pltpu
from jax.experimental.pallas import tpu_sc as plsc

B = 1; T = 2048; D = 1024; H = 16; KV = 8; HD = 64; L = 2; E = 8; TOPK = 2
FF = 2048; V = 32000; EPS = 1e-5; THETA = 1e6

TB = 256
NTB = T // TB
QH = 4
VB = 1280
NVB = V // VB
A = T * TOPK
EB = 256
NEB = A // EB + E
AP = NEB * EB


def _sc_gather(table, idx):
    info = plsc.get_sparse_core_info()
    nc, ns = info.num_cores, info.num_subcores
    nw = nc * ns
    b, d = idx.shape[0], table.shape[1]
    b_per_w = b // nw
    row_bytes = d * table.dtype.itemsize
    nbuf = 2
    max_ch = (460 * 1024) // (nbuf * row_bytes)
    ch = b_per_w
    for cand in (96, 64, 48, 32, 24, 16, 8):
        if cand <= max_ch and b_per_w % cand == 0:
            ch = cand
            break
    n_ch = b_per_w // ch
    nbuf = min(nbuf, n_ch)
    mesh = plsc.VectorSubcoreMesh(core_axis_name="c", subcore_axis_name="s")

    @functools.partial(
        pl.kernel, mesh=mesh,
        out_type=jax.ShapeDtypeStruct((b, d), table.dtype),
        scratch_types=[
            pltpu.VMEM((n_ch, ch), jnp.int32),
            pltpu.VMEM((nbuf, ch, d), table.dtype),
            pltpu.SemaphoreType.DMA,
        ],
    )
    def k(table_hbm, idx_hbm, out_hbm, idx_v, rows_v, sem):
        wid = lax.axis_index("s") * nc + lax.axis_index("c")
        base = wid * b_per_w
        for c in range(n_ch):
            pltpu.sync_copy(idx_hbm.at[pl.ds(base + c * ch, ch)],
                            idx_v.at[c])
        cps = [None] * n_ch
        for c in range(nbuf):
            cps[c] = pltpu.async_copy(table_hbm.at[idx_v.at[c]],
                                      rows_v.at[c % nbuf], sem)
        for c in range(n_ch):
            cps[c].wait()
            pltpu.sync_copy(rows_v.at[c % nbuf],
                            out_hbm.at[pl.ds(base + c * ch, ch)])
            if c + nbuf < n_ch:
                cps[c + nbuf] = pltpu.async_copy(
                    table_hbm.at[idx_v.at[c + nbuf]],
                    rows_v.at[(c + nbuf) % nbuf], sem)

    return k(table, idx)


def _rms_rows(x, w):
    v = jnp.mean(x * x, axis=-1, keepdims=True)
    return x * lax.rsqrt(v + EPS) * w


def _qkv_body(x_ref, ln_ref, wq_ref, wk_ref, wv_ref, q_ref, k_ref, v_ref):
    h = _rms_rows(x_ref[...], ln_ref[0])
    q_ref[...] = jnp.dot(h, wq_ref[0], preferred_element_type=jnp.float32)
    k_ref[...] = jnp.dot(h, wk_ref[0], preferred_element_type=jnp.float32)
    v_ref[...] = jnp.dot(h, wv_ref[0], preferred_element_type=jnp.float32)


def _rope(x, cos, sin):
    h = HD // 2
    rot = jnp.concatenate([-x[:, h:], x[:, :h]], axis=1)
    return x * cos + rot * sin


def _attn_body(q_ref, k_ref, v_ref, cq_ref, sq_ref, ck_ref, sk_ref, o_ref):
    qi = pl.program_id(1)
    row = qi * TB + lax.broadcasted_iota(jnp.int32, (TB, T), 0)
    col = lax.broadcasted_iota(jnp.int32, (TB, T), 1)
    mask = jnp.where(col <= row, 0.0, -1e9)
    kr = [_rope(k_ref[:, j * HD:(j + 1) * HD], ck_ref[...], sk_ref[...])
          for j in range(QH // 2)]
    outs = []
    for j in range(QH):
        q = _rope(q_ref[:, j * HD:(j + 1) * HD], cq_ref[...], sq_ref[...])
        s = lax.dot_general(q, kr[j // 2], (((1,), (1,)), ((), ())),
                            preferred_element_type=jnp.float32)
        s = s * (1.0 / 8.0) + mask
        m = jnp.max(s, axis=1, keepdims=True)
        p = jnp.exp(s - m)
        p = p / jnp.sum(p, axis=1, keepdims=True)
        outs.append(jnp.dot(p, v_ref[:, (j // 2) * HD:(j // 2 + 1) * HD],
                            preferred_element_type=jnp.float32))
    o_ref[...] = jnp.concatenate(outs, axis=1)


def _outproj_body(x_ref, o_ref, wo_ref, y_ref):
    y_ref[...] = x_ref[...] + jnp.dot(o_ref[...], wo_ref[0],
                                      preferred_element_type=jnp.float32)


def _router_body(x_ref, ln_ref, g_ref, h_ref, ti_ref, tv_ref):
    h = _rms_rows(x_ref[...], ln_ref[0])
    h_ref[...] = h
    gl = jnp.dot(h, g_ref[0], preferred_element_type=jnp.float32)
    lane = lax.broadcasted_iota(jnp.int32, (TB, 128), 1)
    gl = jnp.where(lane < E, gl, -1e30)
    m = jnp.max(gl, axis=1, keepdims=True)
    e = jnp.exp(gl - m)
    p = e / jnp.sum(e, axis=1, keepdims=True)
    m1 = jnp.max(p, axis=1, keepdims=True)
    i1 = jnp.min(jnp.where(p == m1, lane, 9999), axis=1, keepdims=True)
    p2 = jnp.where(lane == i1, -1.0, p)
    m2 = jnp.max(p2, axis=1, keepdims=True)
    i2 = jnp.min(jnp.where(p2 == m2, lane, 9999), axis=1, keepdims=True)
    tot = m1 + m2
    ti_ref[...] = jnp.where(lane == 0, i1, jnp.where(lane == 1, i2, 0))
    tv_ref[...] = jnp.where(lane == 0, m1 / tot,
                            jnp.where(lane == 1, m2 / tot, 0.0))


FF2 = FF // 2


def _expert_body(be_ref, h_ref, gi_ref, w1_ref, w2_ref, w3_ref, wv_ref,
                 os_ref, hs_s):
    f = pl.program_id(1)

    @pl.when(f == 0)
    def _():
        cols = lax.broadcasted_iota(jnp.int32, (EB, T), 1)
        oh = (cols == gi_ref[...]).astype(jnp.float32)
        hs_s[...] = jnp.dot(oh, h_ref[...],
                            preferred_element_type=jnp.float32)

    h = hs_s[...]
    a = jax.nn.silu(jnp.dot(h, w1_ref[0, 0],
                            preferred_element_type=jnp.float32))
    a = a * jnp.dot(h, w3_ref[0, 0], preferred_element_type=jnp.float32)
    o = jnp.dot(a, w2_ref[0, 0], preferred_element_type=jnp.float32)

    @pl.when(f == 0)
    def _():
        os_ref[...] = o * wv_ref[...]

    @pl.when(f == 1)
    def _():
        os_ref[...] = os_ref[...] + o * wv_ref[...]


def _add3_body(x_ref, os_ref, p1_ref, p2_ref, y_ref):
    cols = lax.broadcasted_iota(jnp.int32, (TB, AP), 1)
    comb = ((cols == p1_ref[...]) | (cols == p2_ref[...])).astype(
        jnp.float32)
    y_ref[...] = x_ref[...] + jnp.dot(comb, os_ref[...],
                                      preferred_element_type=jnp.float32)


def _ce_body(xf_ref, fn_ref, w_ref, lab_ref, out_ref, m_s, s_s, l_s):
    j = pl.program_id(0)

    @pl.when(j == 0)
    def _():
        m_s[...] = jnp.full((T, 1), -1e30, jnp.float32)
        s_s[...] = jnp.zeros((T, 1), jnp.float32)
        l_s[...] = jnp.zeros((T, 1), jnp.float32)

    xn = _rms_rows(xf_ref[...], fn_ref[...])
    lg = jnp.dot(xn, w_ref[...], preferred_element_type=jnp.float32)
    bm = jnp.max(lg, axis=1, keepdims=True)
    m_old = m_s[...]
    m_new = jnp.maximum(m_old, bm)
    s_s[...] = s_s[...] * jnp.exp(m_old - m_new) + jnp.sum(
        jnp.exp(lg - m_new), axis=1, keepdims=True)
    m_s[...] = m_new
    cols = j * VB + lax.broadcasted_iota(jnp.int32, (T, VB), 1)
    match = cols == lab_ref[...]
    l_s[...] = l_s[...] + jnp.sum(jnp.where(match, lg, 0.0), axis=1,
                                  keepdims=True)

    @pl.when(j == NVB - 1)
    def _():
        lse = m_s[...] + jnp.log(s_s[...])
        out_ref[0, 0] = jnp.mean(lse - l_s[...])


_f32 = jnp.float32


def _qkv_call(l, x, ln, wq, wk, wv):
    return pl.pallas_call(
        _qkv_body,
        grid=(NTB,),
        in_specs=[
            pl.BlockSpec((TB, D), lambda i: (i, 0)),
            pl.BlockSpec((1, 1, D), lambda i: (l, 0, 0)),
            pl.BlockSpec((1, D, H * HD), lambda i: (l, 0, 0)),
            pl.BlockSpec((1, D, KV * HD), lambda i: (l, 0, 0)),
            pl.BlockSpec((1, D, KV * HD), lambda i: (l, 0, 0)),
        ],
        out_specs=[
            pl.BlockSpec((TB, H * HD), lambda i: (i, 0)),
            pl.BlockSpec((TB, KV * HD), lambda i: (i, 0)),
            pl.BlockSpec((TB, KV * HD), lambda i: (i, 0)),
        ],
        out_shape=[
            jax.ShapeDtypeStruct((T, H * HD), _f32),
            jax.ShapeDtypeStruct((T, KV * HD), _f32),
            jax.ShapeDtypeStruct((T, KV * HD), _f32),
        ],
    )(x, ln, wq, wk, wv)


def _attn_call(q, k, v, cos, sin):
    return pl.pallas_call(
        _attn_body,
        grid=(H // QH, NTB),
        in_specs=[
            pl.BlockSpec((TB, QH * HD), lambda h4, qi: (qi, h4)),
            pl.BlockSpec((T, QH * HD // 2), lambda h4, qi: (0, h4)),
            pl.BlockSpec((T, QH * HD // 2), lambda h4, qi: (0, h4)),
            pl.BlockSpec((TB, HD), lambda h4, qi: (qi, 0)),
            pl.BlockSpec((TB, HD), lambda h4, qi: (qi, 0)),
            pl.BlockSpec((T, HD), lambda h4, qi: (0, 0)),
            pl.BlockSpec((T, HD), lambda h4, qi: (0, 0)),
        ],
        out_specs=pl.BlockSpec((TB, QH * HD), lambda h4, qi: (qi, h4)),
        out_shape=jax.ShapeDtypeStruct((T, H * HD), _f32),
    )(q, k, v, cos, sin, cos, sin)


def _outproj_call(l, x, o, wo):
    return pl.pallas_call(
        _outproj_body,
        grid=(NTB,),
        in_specs=[
            pl.BlockSpec((TB, D), lambda i: (i, 0)),
            pl.BlockSpec((TB, H * HD), lambda i: (i, 0)),
            pl.BlockSpec((1, H * HD, D), lambda i: (l, 0, 0)),
        ],
        out_specs=pl.BlockSpec((TB, D), lambda i: (i, 0)),
        out_shape=jax.ShapeDtypeStruct((T, D), _f32),
    )(x, o, wo)


def _router_call(l, x, ln, gate_pad):
    return pl.pallas_call(
        _router_body,
        grid=(NTB,),
        in_specs=[
            pl.BlockSpec((TB, D), lambda i: (i, 0)),
            pl.BlockSpec((1, 1, D), lambda i: (l, 0, 0)),
            pl.BlockSpec((1, D, 128), lambda i: (l, 0, 0)),
        ],
        out_specs=[
            pl.BlockSpec((TB, D), lambda i: (i, 0)),
            pl.BlockSpec((TB, 128), lambda i: (i, 0)),
            pl.BlockSpec((TB, 128), lambda i: (i, 0)),
        ],
        out_shape=[
            jax.ShapeDtypeStruct((T, D), _f32),
            jax.ShapeDtypeStruct((T, 128), jnp.int32),
            jax.ShapeDtypeStruct((T, 128), _f32),
        ],
    )(x, ln, gate_pad)


def _expert_call(l, be, h, gidx, w1, w2, w3, wvec):
    spec = pltpu.PrefetchScalarGridSpec(
        num_scalar_prefetch=1,
        grid=(NEB, 2),
        in_specs=[
            pl.BlockSpec((T, D), lambda b, f, be_ref: (0, 0)),
            pl.BlockSpec((EB, 1), lambda b, f, be_ref: (b, 0)),
            pl.BlockSpec((1, 1, D, FF2),
                         lambda b, f, be_ref: (l, be_ref[b], 0, f)),
            pl.BlockSpec((1, 1, FF2, D),
                         lambda b, f, be_ref: (l, be_ref[b], f, 0)),
            pl.BlockSpec((1, 1, D, FF2),
                         lambda b, f, be_ref: (l, be_ref[b], 0, f)),
            pl.BlockSpec((EB, 1), lambda b, f, be_ref: (b, 0)),
        ],
        out_specs=pl.BlockSpec((EB, D), lambda b, f, be_ref: (b, 0)),
        scratch_shapes=[pltpu.VMEM((EB, D), jnp.float32)],
    )
    return pl.pallas_call(
        _expert_body,
        grid_spec=spec,
        out_shape=jax.ShapeDtypeStruct((AP, D), _f32),
        compiler_params=pltpu.CompilerParams(
            dimension_semantics=("arbitrary", "arbitrary")),
    )(be, h, gidx, w1, w2, w3, wvec)


def _add3_call(x, os_, p1, p2):
    return pl.pallas_call(
        _add3_body,
        grid=(NTB,),
        in_specs=[
            pl.BlockSpec((TB, D), lambda i: (i, 0)),
            pl.BlockSpec((AP, D), lambda i: (0, 0)),
            pl.BlockSpec((TB, 1), lambda i: (i, 0)),
            pl.BlockSpec((TB, 1), lambda i: (i, 0)),
        ],
        out_specs=pl.BlockSpec((TB, D), lambda i: (i, 0)),
        out_shape=jax.ShapeDtypeStruct((T, D), _f32),
    )(x, os_, p1, p2)


def _ce_call(xf, fnorm, lm_head, labels):
    out = pl.pallas_call(
        _ce_body,
        grid=(NVB,),
        in_specs=[
            pl.BlockSpec((T, D), lambda j: (0, 0)),
            pl.BlockSpec((1, D), lambda j: (0, 0)),
            pl.BlockSpec((D, VB), lambda j: (0, j)),
            pl.BlockSpec((T, 1), lambda j: (0, 0)),
        ],
        out_specs=pl.BlockSpec(memory_space=pltpu.SMEM),
        out_shape=jax.ShapeDtypeStruct((1, 1), _f32),
        scratch_shapes=[
            pltpu.VMEM((T, 1), _f32),
            pltpu.VMEM((T, 1), _f32),
            pltpu.VMEM((T, 1), _f32),
        ],
        compiler_params=pltpu.CompilerParams(
            dimension_semantics=("arbitrary",)),
    )(xf, fnorm, lm_head, labels)
    return out[0, 0]


def _route_plan(ti, tv):
    er = jnp.arange(E)
    oh1 = (ti[:, 0:1] == er[None, :]).astype(jnp.int32)
    oh2 = (ti[:, 1:2] == er[None, :]).astype(jnp.int32)
    c1 = jnp.cumsum(oh1, axis=0)
    c2 = jnp.cumsum(oh2, axis=0)
    cnt1 = c1[-1]
    cnt = cnt1 + c2[-1]
    pc = ((cnt + EB - 1) // EB) * EB
    po = jnp.cumsum(pc) - pc
    r1 = jnp.sum((c1 - oh1 + po[None, :]) * oh1, axis=1)
    r2 = jnp.sum((c2 - oh2 + cnt1[None, :] + po[None, :]) * oh2, axis=1)
    tok = jnp.arange(T, dtype=jnp.int32)
    gidx = jnp.zeros((AP,), jnp.int32).at[r1].set(tok).at[r2].set(tok)
    wvec = jnp.zeros((AP,), jnp.float32).at[r1].set(tv[:, 0]).at[r2].set(
        tv[:, 1])
    p1 = r1.reshape(T, 1).astype(jnp.int32)
    p2 = r2.reshape(T, 1).astype(jnp.int32)

    cum_pc = jnp.cumsum(pc)
    s = jnp.arange(NEB) * EB
    be = jnp.minimum(jnp.sum(
        (s[:, None] >= cum_pc[None, :]).astype(jnp.int32), axis=1),
        E - 1).astype(jnp.int32)
    return gidx.reshape(AP, 1), wvec.reshape(AP, 1), p1, p2, be


def kernel(input_ids, labels, embed, ln1, ln2, final_norm, wq, wk, wv, wo,
           gate, w1, w2, w3, lm_head):
    ids = input_ids.reshape(T).astype(jnp.int32)
    x = _sc_gather(embed, ids)

    pos = jnp.arange(T, dtype=jnp.float32)
    inv_f = 1.0 / (THETA ** (jnp.arange(0, HD, 2, dtype=jnp.float32) / HD))
    fr = pos[:, None] * inv_f[None, :]
    emb = jnp.concatenate([fr, fr], axis=-1)
    cos = jnp.cos(emb)
    sin = jnp.sin(emb)

    gate_pad = jnp.pad(gate, ((0, 0), (0, 0), (0, 128 - E)))
    ln1 = ln1.reshape(L, 1, D)
    ln2 = ln2.reshape(L, 1, D)
    for l in range(L):
        q, k, v = _qkv_call(l, x, ln1, wq, wk, wv)
        o = _attn_call(q, k, v, cos, sin)
        x = _outproj_call(l, x, o, wo)

        h, ti_p, tv_p = _router_call(l, x, ln2, gate_pad)
        ti = ti_p[:, :TOPK]
        tv = tv_p[:, :TOPK]
        gidx, wvec, p1, p2, be = _route_plan(ti, tv)

        os_ = _expert_call(l, be, h, gidx, w1, w2, w3, wvec)
        x = _add3_call(x, os_, p1, p2)

    lab = labels.reshape(T, 1).astype(jnp.int32)
    return _ce_call(x, final_norm.reshape(1, D), lm_head, lab)
